# baseline (speedup 1.0000x reference)
import functools

import jax
import jax.numpy as jnp
from jax.experimental import pallas as pl
from jax.experimental.pallas import tpu as pltpu

_B, _V, _S = 16, 64, 2048
_SAMPLE_DIM, _PATCH_DIM = 128, 256
_WINDOW, _STEP = 32, 16
_C = _S // _STEP
_T = (_S - _WINDOW + _STEP - 1) // _STEP
_K2 = _STEP * _SAMPLE_DIM
_BB = 4


def _tok_kernel(x_ref, As_ref, bs_ref, Ap_ref, bp_ref, lnw_ref,
                lnb_ref, o_ref, A12_ref):
    As = As_ref[...]
    pres = [
        jax.lax.dot_general(x_ref[i], As, (((0,), (0,)), ((), ())),
                            preferred_element_type=jnp.float32)
        for i in range(_BB)
    ]
    pre = jnp.concatenate(pres, axis=0)
    enc = jnp.floor(pre + bs_ref[...].reshape(1, _SAMPLE_DIM))
    E = enc.astype(jnp.bfloat16).reshape(_BB * _C, _K2)

    @pl.when(pl.program_id(0) == 0)
    def _build_a12():
        ap = Ap_ref[...].reshape(_SAMPLE_DIM, _WINDOW, _PATCH_DIM)
        full = ap.swapaxes(0, 1).reshape(_WINDOW * _SAMPLE_DIM, _PATCH_DIM)
        A12_ref[...] = jnp.concatenate(
            [full[:_K2], full[_K2:]], axis=1).astype(jnp.bfloat16)

    t12 = jnp.dot(E, A12_ref[...],
                  preferred_element_type=jnp.float32)
    t2s = jnp.roll(t12[:, _PATCH_DIM:], -1, axis=0)
    tok = jnp.floor(t12[:, :_PATCH_DIM] + t2s
                    + bp_ref[...].reshape(1, _PATCH_DIM))
    m = jnp.mean(tok, axis=1, keepdims=True)
    v = jnp.mean(tok * tok, axis=1, keepdims=True) - m * m
    o = ((tok - m) * jax.lax.rsqrt(v + 1e-5)
         * lnw_ref[...].reshape(1, _PATCH_DIM)
         + lnb_ref[...].reshape(1, _PATCH_DIM))
    for i in range(_BB):
        o_ref[i] = o[i * _C:i * _C + _T]


@functools.partial(jax.jit, static_argnames=("interpret",))
def kernel(x, A_sample, b_sample, A_patch, b_patch, ln_weight, ln_bias,
           interpret=False):
    return pl.pallas_call(
        _tok_kernel,
        grid=(_B // _BB,),
        in_specs=[
            pl.BlockSpec((_BB, _V, _S), lambda b: (b, 0, 0)),
            pl.BlockSpec((_V, _SAMPLE_DIM), lambda b: (0, 0)),
            pl.BlockSpec((_SAMPLE_DIM,), lambda b: (0,)),
            pl.BlockSpec((_WINDOW * _SAMPLE_DIM, _PATCH_DIM),
                         lambda b: (0, 0)),
            pl.BlockSpec((_PATCH_DIM,), lambda b: (0,)),
            pl.BlockSpec((_PATCH_DIM,), lambda b: (0,)),
            pl.BlockSpec((_PATCH_DIM,), lambda b: (0,)),
        ],
        out_specs=pl.BlockSpec((_BB, _T, _PATCH_DIM), lambda b: (b, 0, 0)),
        out_shape=jax.ShapeDtypeStruct((_B, _T, _PATCH_DIM), jnp.float32),
        scratch_shapes=[pltpu.VMEM((_K2, 2 * _PATCH_DIM), jnp.bfloat16)],
        interpret=interpret,
    )(x, A_sample, b_sample, A_patch, b_patch, ln_weight, ln_bias)


# speedup vs baseline: 1.0562x; 1.0562x over previous
import functools

import jax
import jax.numpy as jnp
from jax.experimental import pallas as pl
from jax.experimental.pallas import tpu as pltpu

_B, _V, _S = 16, 64, 2048
_SAMPLE_DIM, _PATCH_DIM = 128, 256
_WINDOW, _STEP = 32, 16
_C = _S // _STEP
_T = (_S - _WINDOW + _STEP - 1) // _STEP
_K2 = _STEP * _SAMPLE_DIM
_BB = 4


def _tok_kernel(x_ref, As_ref, bs_ref, Ap_ref, bp_ref, lnw_ref,
                lnb_ref, o_ref, A12_ref):
    @pl.when(pl.program_id(0) == 0)
    def _build_a12():
        ap = Ap_ref[...].reshape(_SAMPLE_DIM, _WINDOW, _PATCH_DIM)
        full = ap.swapaxes(0, 1).reshape(_WINDOW * _SAMPLE_DIM, _PATCH_DIM)
        A12_ref[...] = jnp.concatenate(
            [full[:_K2], full[_K2:]], axis=1).astype(jnp.bfloat16)

    As = As_ref[...]
    pres = [
        jax.lax.dot_general(x_ref[i], As, (((0,), (0,)), ((), ())),
                            preferred_element_type=jnp.float32)
        for i in range(_BB)
    ]
    pre = jnp.concatenate(pres, axis=0)
    enc = jnp.floor(pre + bs_ref[...].reshape(1, _SAMPLE_DIM))
    E = enc.astype(jnp.bfloat16).reshape(_BB * _C, _K2)
    t12 = jnp.dot(E, A12_ref[...],
                  preferred_element_type=jnp.float32)
    t2s = jnp.roll(t12[:, _PATCH_DIM:], -1, axis=0)
    tok = jnp.floor(t12[:, :_PATCH_DIM] + t2s
                    + bp_ref[...].reshape(1, _PATCH_DIM))
    m = jnp.mean(tok, axis=1, keepdims=True)
    v = jnp.mean(tok * tok, axis=1, keepdims=True) - m * m
    o = ((tok - m) * jax.lax.rsqrt(v + 1e-5)
         * lnw_ref[...].reshape(1, _PATCH_DIM)
         + lnb_ref[...].reshape(1, _PATCH_DIM))
    for i in range(_BB):
        o_ref[i] = o[i * _C:i * _C + _T]


@functools.partial(jax.jit, static_argnames=("interpret",))
def kernel(x, A_sample, b_sample, A_patch, b_patch, ln_weight, ln_bias,
           interpret=False):
    return pl.pallas_call(
        _tok_kernel,
        grid=(_B // _BB,),
        in_specs=[
            pl.BlockSpec((_BB, _V, _S), lambda b: (b, 0, 0)),
            pl.BlockSpec((_V, _SAMPLE_DIM), lambda b: (0, 0)),
            pl.BlockSpec((_SAMPLE_DIM,), lambda b: (0,)),
            pl.BlockSpec((_WINDOW * _SAMPLE_DIM, _PATCH_DIM),
                         lambda b: (0, 0)),
            pl.BlockSpec((_PATCH_DIM,), lambda b: (0,)),
            pl.BlockSpec((_PATCH_DIM,), lambda b: (0,)),
            pl.BlockSpec((_PATCH_DIM,), lambda b: (0,)),
        ],
        out_specs=pl.BlockSpec((_BB, _T, _PATCH_DIM), lambda b: (b, 0, 0)),
        out_shape=jax.ShapeDtypeStruct((_B, _T, _PATCH_DIM), jnp.float32),
        scratch_shapes=[pltpu.VMEM((_K2, 2 * _PATCH_DIM), jnp.bfloat16)],
        interpret=interpret,
    )(x, A_sample, b_sample, A_patch, b_patch, ln_weight, ln_bias)
